# baseline (speedup 1.0000x reference)
import functools

import jax
import jax.numpy as jnp
from jax import lax
from jax.experimental import pallas as pl
from jax.experimental.pallas import tpu as pltpu
from jax.experimental.pallas import tpu_sc as plsc

N = 10000
E = 320000
D_IN = 128
D_HID = 256
D_OUT = 128

NC = 2
NS = 16
NW = NC * NS

CH = 128
NP = 10240
ROWS_PER_TILE = NP // NS
EP = 327680
FB = 128

_mesh = plsc.VectorSubcoreMesh(core_axis_name="c", subcore_axis_name="s")


def _deg_body(dst_hbm, zeros_hbm, out_hbm, dst_v, ones_v, acc, _sem):
    c = lax.axis_index("c")
    s = lax.axis_index("s")
    r0 = s * ROWS_PER_TILE
    pltpu.sync_copy(zeros_hbm.at[pl.ds(r0, ROWS_PER_TILE)],
                    acc.at[pl.ds(r0, ROWS_PER_TILE)])
    for i in range(CH // 16):
        ones_v[pl.ds(i * 16, 16)] = jnp.ones((16,), jnp.float32)

    nch = EP // NW // CH
    ch0 = (c * NS + s) * nch
    pltpu.sync_copy(dst_hbm.at[pl.ds(ch0, nch)], dst_v)
    plsc.subcore_barrier()

    def body(i, carry):
        pltpu.sync_copy(ones_v, acc.at[dst_v.at[i]], add=True)
        return carry

    lax.fori_loop(0, nch, body, 0)
    plsc.subcore_barrier()
    pltpu.sync_copy(acc.at[pl.ds(r0, ROWS_PER_TILE)],
                    out_hbm.at[c, pl.ds(r0, ROWS_PER_TILE)])


_deg_call = pl.kernel(
    _deg_body,
    out_type=jax.ShapeDtypeStruct((NC, NP), jnp.float32),
    mesh=_mesh,
    scratch_types=[
        pltpu.VMEM((EP // NW // CH, CH), jnp.int32),
        pltpu.VMEM((CH,), jnp.float32),
        pltpu.VMEM_SHARED((NP,), jnp.float32),
        pltpu.SemaphoreType.DMA,
    ],
)


def _agg_body(src_hbm, dst_hbm, g_hbm, zeros_hbm, out_hbm,
              src0, src1, dst0, dst1, rows0, rows1, stbl, acc,
              gsem0, gsem1, isem0, isem1, *, ncb):
    srcb = [src0, src1]
    dstb = [dst0, dst1]
    rows = [rows0, rows1]
    gsem = [gsem0, gsem1]
    isem = [isem0, isem1]
    c = lax.axis_index("c")
    s = lax.axis_index("s")
    r0 = s * ROWS_PER_TILE
    rpt = pl.ds(r0, ROWS_PER_TILE)
    nch = EP // NS // CH
    ch0 = s * nch
    npass = ncb // NC

    kb = c % ncb
    pltpu.sync_copy(g_hbm.at[kb].at[rpt], stbl.at[rpt])
    pltpu.sync_copy(zeros_hbm.at[pl.ds(s * 80, 80)], acc.at[pl.ds(s * 80, 80)])
    plsc.subcore_barrier()

    def body(i, carry):
        pltpu.sync_copy(src_hbm.at[ch0 + i], srcb[0])
        pltpu.sync_copy(dst_hbm.at[ch0 + i], dstb[0])
        pltpu.async_copy(stbl.at[srcb[0]], rows[0], gsem[0]).wait()
        pltpu.sync_copy(rows[0], acc.at[dstb[0]], add=True)
        return carry

    lax.fori_loop(0, nch, body, 0)
    plsc.subcore_barrier()
    pltpu.sync_copy(acc.at[pl.ds(s * 80, 80)],
                    out_hbm.at[kb].at[pl.ds(s * 80, 80)])


def _make_agg(ncb):
    return pl.kernel(
        functools.partial(_agg_body, ncb=ncb),
        out_type=jax.ShapeDtypeStruct((ncb, 1280, FB), jnp.float32),
        mesh=_mesh,
        scratch_types=[
            pltpu.VMEM((CH,), jnp.int32),
            pltpu.VMEM((CH,), jnp.int32),
            pltpu.VMEM((CH,), jnp.int32),
            pltpu.VMEM((CH,), jnp.int32),
            pltpu.VMEM((CH, 128), jnp.float32),
            pltpu.VMEM((CH, 128), jnp.float32),
            pltpu.VMEM_SHARED((NP, 128), jnp.float32),
            pltpu.VMEM_SHARED((1280, 128), jnp.float32),
            pltpu.SemaphoreType.DMA,
            pltpu.SemaphoreType.DMA,
            pltpu.SemaphoreType.DMA,
            pltpu.SemaphoreType.DMA,
        ],
    )


_agg2 = _make_agg(2)
_agg1 = _make_agg(1)


def _tc_b(embed_ref, w1_ref, degt_ref, g1_ref, dinv_ref):
    degt = degt_ref[...]
    deg = degt[:, 0:1] + degt[:, 1:2] + 1.0
    dinv = lax.rsqrt(deg)
    h = jnp.dot(embed_ref[...], w1_ref[...],
                preferred_element_type=jnp.float32)
    g = h * dinv
    for k in range(D_HID // FB):
        g1_ref[k] = g[:, k * FB:(k + 1) * FB]
    dinv_ref[...] = dinv


def _tc_d(res1_ref, g1_ref, dinv_ref, b1_ref, w2_ref, g2_ref):
    dinv = dinv_ref[...]
    b1 = b1_ref[...]
    w2 = w2_ref[...]
    acc = jnp.zeros((res1_ref.shape[1], D_OUT), jnp.float32)
    for k in range(D_HID // FB):
        hk = (res1_ref[k] + g1_ref[k]) * dinv + b1[:, k * FB:(k + 1) * FB]
        acc = acc + jnp.dot(hk, w2[k * FB:(k + 1) * FB],
                            preferred_element_type=jnp.float32)
    g2 = acc * dinv
    for k in range(D_OUT // FB):
        g2_ref[k] = g2[:, k * FB:(k + 1) * FB]


def _tc_f(res2_ref, g2_ref, dinv_ref, b2_ref, gamma_ref, beta_ref, out_ref):
    o = jnp.concatenate(
        [res2_ref[k] + g2_ref[k] for k in range(D_OUT // FB)], axis=1)
    o = o * dinv_ref[...] + b2_ref[...]
    rowid = lax.broadcasted_iota(jnp.int32, (NP, 1), 0)
    mask = (rowid < N).astype(jnp.float32)
    mu = jnp.sum(o * mask, axis=0, keepdims=True) * (1.0 / N)
    d = (o - mu) * mask
    var = jnp.sum(d * d, axis=0, keepdims=True) * (1.0 / N)
    y = (o - mu) * lax.rsqrt(var + 1e-5) * gamma_ref[...] + beta_ref[...]
    out_ref[...] = y[:N]


_RB = 2048

_tc_b_call = pl.pallas_call(
    _tc_b,
    grid=(NP // _RB,),
    in_specs=[pl.BlockSpec((_RB, D_IN), lambda i: (i, 0)),
              pl.BlockSpec((D_IN, D_HID), lambda i: (0, 0)),
              pl.BlockSpec((_RB, NC), lambda i: (i, 0))],
    out_specs=(pl.BlockSpec((D_HID // FB, _RB, FB), lambda i: (0, i, 0)),
               pl.BlockSpec((_RB, 1), lambda i: (i, 0))),
    out_shape=(jax.ShapeDtypeStruct((D_HID // FB, NP, FB), jnp.float32),
               jax.ShapeDtypeStruct((NP, 1), jnp.float32)),
)

_tc_d_call = pl.pallas_call(
    _tc_d,
    grid=(NP // _RB,),
    in_specs=[pl.BlockSpec((D_HID // FB, _RB, FB), lambda i: (0, i, 0)),
              pl.BlockSpec((D_HID // FB, _RB, FB), lambda i: (0, i, 0)),
              pl.BlockSpec((_RB, 1), lambda i: (i, 0)),
              pl.BlockSpec((1, D_HID), lambda i: (0, 0)),
              pl.BlockSpec((D_HID, D_OUT), lambda i: (0, 0))],
    out_specs=pl.BlockSpec((D_OUT // FB, _RB, FB), lambda i: (0, i, 0)),
    out_shape=jax.ShapeDtypeStruct((D_OUT // FB, NP, FB), jnp.float32),
)

_tc_f_call = pl.pallas_call(
    _tc_f,
    out_shape=jax.ShapeDtypeStruct((N, D_OUT), jnp.float32),
)


@jax.jit
def kernel(embed, edge_index, W1, b1, W2, b2, gamma, beta):
    src = edge_index[0]
    dst = edge_index[1]
    pad_idx = jnp.full((EP - E,), N, dtype=jnp.int32)
    src_p = jnp.concatenate([src, pad_idx]).reshape(EP // CH, CH)
    dst_p = jnp.concatenate([dst, pad_idx]).reshape(EP // CH, CH)
    embed_p = jnp.pad(embed, ((0, NP - N), (0, 0)))
    zeros1 = jnp.zeros((NP,), jnp.float32)
    zeros2 = jnp.zeros((NP, FB), jnp.float32)

    dst_m = jnp.mod(dst_p, 1280)

    degs = _deg_call(dst_p, zeros1)
    degt = jnp.transpose(degs)

    g1, dinv = _tc_b_call(embed_p, W1, degt)
    res1t = _agg2(src_p, dst_m, g1, zeros2)
    res1 = jnp.pad(res1t, ((0, 0), (0, NP - 1280), (0, 0)))
    g2 = _tc_d_call(res1, g1, dinv, b1.reshape(1, D_HID), W2)
    res2t = _agg1(src_p, dst_m, g2, zeros2)
    res2 = jnp.pad(res2t, ((0, 0), (0, NP - 1280), (0, 0)))
    res2 = jnp.concatenate([res2, res2], axis=0)
    out = _tc_f_call(res2, g2, dinv, b2.reshape(1, D_OUT),
                     gamma.reshape(1, D_OUT), beta.reshape(1, D_OUT))
    return out


# speedup vs baseline: 1.2288x; 1.2288x over previous
import functools

import jax
import jax.numpy as jnp
from jax import lax
from jax.experimental import pallas as pl
from jax.experimental.pallas import tpu as pltpu
from jax.experimental.pallas import tpu_sc as plsc

N = 10000
E = 320000
D_IN = 128
D_HID = 256
D_OUT = 128

NC = 2
NS = 16
NW = NC * NS

CH = 128
NP = 10240
ROWS_PER_TILE = NP // NS
EP = 327680
FB = 128

_mesh = plsc.VectorSubcoreMesh(core_axis_name="c", subcore_axis_name="s")


def _deg_body(dst_hbm, zeros_hbm, out_hbm, dst_v, ones_v, acc, _sem):
    c = lax.axis_index("c")
    s = lax.axis_index("s")
    r0 = s * ROWS_PER_TILE
    pltpu.sync_copy(zeros_hbm.at[pl.ds(r0, ROWS_PER_TILE)],
                    acc.at[pl.ds(r0, ROWS_PER_TILE)])
    for i in range(CH // 16):
        ones_v[pl.ds(i * 16, 16)] = jnp.ones((16,), jnp.float32)

    nch = EP // NW // CH
    ch0 = (c * NS + s) * nch
    pltpu.sync_copy(dst_hbm.at[pl.ds(ch0, nch)], dst_v)
    plsc.subcore_barrier()

    def body(i, carry):
        pltpu.sync_copy(ones_v, acc.at[dst_v.at[i]], add=True)
        return carry

    lax.fori_loop(0, nch, body, 0)
    plsc.subcore_barrier()
    pltpu.sync_copy(acc.at[pl.ds(r0, ROWS_PER_TILE)],
                    out_hbm.at[c, pl.ds(r0, ROWS_PER_TILE)])


_deg_call = pl.kernel(
    _deg_body,
    out_type=jax.ShapeDtypeStruct((NC, NP), jnp.float32),
    mesh=_mesh,
    scratch_types=[
        pltpu.VMEM((EP // NW // CH, CH), jnp.int32),
        pltpu.VMEM((CH,), jnp.float32),
        pltpu.VMEM_SHARED((NP,), jnp.float32),
        pltpu.SemaphoreType.DMA,
    ],
)


def _gat_body(src_hbm, g_hbm, er_hbm, src0, src1, rows0, rows1, stbl,
              gsem0, gsem1, isem0, isem1, *, ncb):
    srcb = [src0, src1]
    rows = [rows0, rows1]
    gsem = [gsem0, gsem1]
    isem = [isem0, isem1]
    c = lax.axis_index("c")
    s = lax.axis_index("s")
    rpt = pl.ds(s * ROWS_PER_TILE, ROWS_PER_TILE)
    if ncb == 2:
        nch = EP // NS // CH
        ch0 = s * nch
        kb = c
    else:
        nch = EP // NW // CH
        ch0 = (c * NS + s) * nch
        kb = 0
    pltpu.sync_copy(g_hbm.at[kb].at[rpt], stbl.at[rpt])
    plsc.subcore_barrier()

    pltpu.sync_copy(src_hbm.at[ch0], srcb[0])
    pltpu.async_copy(src_hbm.at[ch0 + 1], srcb[1], isem[1])
    pltpu.async_copy(stbl.at[srcb[0]], rows[0], gsem[0])

    def body(jj, carry):
        for b in range(2):
            ch = jj * 2 + b
            o = 1 - b
            pltpu.make_async_copy(stbl.at[srcb[b]], rows[b], gsem[b]).wait()

            @pl.when(ch + 1 < nch)
            def _():
                pltpu.make_async_copy(src_hbm.at[ch0], srcb[o], isem[o]).wait()
                pltpu.async_copy(stbl.at[srcb[o]], rows[o], gsem[o])

            pltpu.sync_copy(rows[b],
                            er_hbm.at[kb].at[pl.ds((ch0 + ch) * CH, CH)])

            @pl.when(ch + 2 < nch)
            def _():
                pltpu.async_copy(src_hbm.at[ch0 + ch + 2], srcb[b], isem[b])
        return carry

    lax.fori_loop(0, nch // 2, body, 0)


def _make_gat(ncb):
    return pl.kernel(
        functools.partial(_gat_body, ncb=ncb),
        out_type=jax.ShapeDtypeStruct((ncb, EP, FB), jnp.float32),
        mesh=_mesh,
        scratch_types=[
            pltpu.VMEM((CH,), jnp.int32),
            pltpu.VMEM((CH,), jnp.int32),
            pltpu.VMEM((CH, FB), jnp.float32),
            pltpu.VMEM((CH, FB), jnp.float32),
            pltpu.VMEM_SHARED((NP, FB), jnp.float32),
            pltpu.SemaphoreType.DMA,
            pltpu.SemaphoreType.DMA,
            pltpu.SemaphoreType.DMA,
            pltpu.SemaphoreType.DMA,
        ],
    )


def _sca_body(dst_hbm, er_hbm, zeros_hbm, out_hbm,
              dst0, dst1, rows0, rows1, acc,
              rsem0, rsem1, isem0, isem1, *, ncb):
    dstb = [dst0, dst1]
    rows = [rows0, rows1]
    rsem = [rsem0, rsem1]
    isem = [isem0, isem1]
    c = lax.axis_index("c")
    s = lax.axis_index("s")
    rpt = pl.ds(s * ROWS_PER_TILE, ROWS_PER_TILE)
    if ncb == 2:
        nch = EP // NS // CH
        ch0 = s * nch
        kb = c
    else:
        nch = EP // NW // CH
        ch0 = (c * NS + s) * nch
        kb = 0
    pltpu.sync_copy(zeros_hbm.at[rpt], acc.at[rpt])
    plsc.subcore_barrier()

    def _er(ch):
        return er_hbm.at[kb].at[pl.ds((ch0 + ch) * CH, CH)]

    pltpu.sync_copy(dst_hbm.at[ch0], dstb[0])
    pltpu.async_copy(dst_hbm.at[ch0 + 1], dstb[1], isem[1])
    pltpu.async_copy(_er(0), rows[0], rsem[0])

    def body(jj, carry):
        for b in range(2):
            ch = jj * 2 + b
            o = 1 - b
            pltpu.make_async_copy(_er(ch), rows[b], rsem[b]).wait()

            @pl.when(ch + 1 < nch)
            def _():
                pltpu.async_copy(_er(ch + 1), rows[o], rsem[o])

            @pl.when(ch >= 1)
            def _():
                pltpu.make_async_copy(dst_hbm.at[ch0], dstb[b], isem[b]).wait()

            pltpu.sync_copy(rows[b], acc.at[dstb[b]], add=True)

            @pl.when(ch + 2 < nch)
            def _():
                pltpu.async_copy(dst_hbm.at[ch0 + ch + 2], dstb[b], isem[b])
        return carry

    lax.fori_loop(0, nch // 2, body, 0)
    plsc.subcore_barrier()
    pltpu.sync_copy(acc.at[rpt], out_hbm.at[c].at[rpt])


def _make_sca(ncb):
    return pl.kernel(
        functools.partial(_sca_body, ncb=ncb),
        out_type=jax.ShapeDtypeStruct((NC, NP, FB), jnp.float32),
        mesh=_mesh,
        scratch_types=[
            pltpu.VMEM((CH,), jnp.int32),
            pltpu.VMEM((CH,), jnp.int32),
            pltpu.VMEM((CH, FB), jnp.float32),
            pltpu.VMEM((CH, FB), jnp.float32),
            pltpu.VMEM_SHARED((NP, FB), jnp.float32),
            pltpu.SemaphoreType.DMA,
            pltpu.SemaphoreType.DMA,
            pltpu.SemaphoreType.DMA,
            pltpu.SemaphoreType.DMA,
        ],
    )


_gat2 = _make_gat(2)
_gat1 = _make_gat(1)
_sca2 = _make_sca(2)
_sca1 = _make_sca(1)


def _tc_b(embed_ref, w1_ref, degt_ref, g1_ref, dinv_ref):
    degt = degt_ref[...]
    deg = degt[:, 0:1] + degt[:, 1:2] + 1.0
    dinv = lax.rsqrt(deg)
    h = jnp.dot(embed_ref[...], w1_ref[...],
                preferred_element_type=jnp.float32)
    g = h * dinv
    for k in range(D_HID // FB):
        g1_ref[k] = g[:, k * FB:(k + 1) * FB]
    dinv_ref[...] = dinv


def _tc_d(res1_ref, g1_ref, dinv_ref, b1_ref, w2_ref, g2_ref):
    dinv = dinv_ref[...]
    b1 = b1_ref[...]
    w2 = w2_ref[...]
    acc = jnp.zeros((res1_ref.shape[1], D_OUT), jnp.float32)
    for k in range(D_HID // FB):
        hk = (res1_ref[k] + g1_ref[k]) * dinv + b1[:, k * FB:(k + 1) * FB]
        acc = acc + jnp.dot(hk, w2[k * FB:(k + 1) * FB],
                            preferred_element_type=jnp.float32)
    g2 = acc * dinv
    for k in range(D_OUT // FB):
        g2_ref[k] = g2[:, k * FB:(k + 1) * FB]


def _tc_f(res2_ref, g2_ref, dinv_ref, b2_ref, gamma_ref, beta_ref, out_ref):
    o = res2_ref[0] + res2_ref[1] + g2_ref[0]
    o = o * dinv_ref[...] + b2_ref[...]
    rowid = lax.broadcasted_iota(jnp.int32, (NP, 1), 0)
    mask = (rowid < N).astype(jnp.float32)
    mu = jnp.sum(o * mask, axis=0, keepdims=True) * (1.0 / N)
    d = (o - mu) * mask
    var = jnp.sum(d * d, axis=0, keepdims=True) * (1.0 / N)
    y = (o - mu) * lax.rsqrt(var + 1e-5) * gamma_ref[...] + beta_ref[...]
    out_ref[...] = y[:N]


_RB = 2048

_tc_b_call = pl.pallas_call(
    _tc_b,
    grid=(NP // _RB,),
    in_specs=[pl.BlockSpec((_RB, D_IN), lambda i: (i, 0)),
              pl.BlockSpec((D_IN, D_HID), lambda i: (0, 0)),
              pl.BlockSpec((_RB, NC), lambda i: (i, 0))],
    out_specs=(pl.BlockSpec((D_HID // FB, _RB, FB), lambda i: (0, i, 0)),
               pl.BlockSpec((_RB, 1), lambda i: (i, 0))),
    out_shape=(jax.ShapeDtypeStruct((D_HID // FB, NP, FB), jnp.float32),
               jax.ShapeDtypeStruct((NP, 1), jnp.float32)),
)

_tc_d_call = pl.pallas_call(
    _tc_d,
    grid=(NP // _RB,),
    in_specs=[pl.BlockSpec((D_HID // FB, _RB, FB), lambda i: (0, i, 0)),
              pl.BlockSpec((D_HID // FB, _RB, FB), lambda i: (0, i, 0)),
              pl.BlockSpec((_RB, 1), lambda i: (i, 0)),
              pl.BlockSpec((1, D_HID), lambda i: (0, 0)),
              pl.BlockSpec((D_HID, D_OUT), lambda i: (0, 0))],
    out_specs=pl.BlockSpec((D_OUT // FB, _RB, FB), lambda i: (0, i, 0)),
    out_shape=jax.ShapeDtypeStruct((D_OUT // FB, NP, FB), jnp.float32),
)

_tc_f_call = pl.pallas_call(
    _tc_f,
    out_shape=jax.ShapeDtypeStruct((N, D_OUT), jnp.float32),
)


@jax.jit
def kernel(embed, edge_index, W1, b1, W2, b2, gamma, beta):
    src = edge_index[0]
    dst = edge_index[1]
    pad_idx = jnp.full((EP - E,), N, dtype=jnp.int32)
    src_p = jnp.concatenate([src, pad_idx]).reshape(EP // CH, CH)
    dst_p = jnp.concatenate([dst, pad_idx]).reshape(EP // CH, CH)
    embed_p = jnp.pad(embed, ((0, NP - N), (0, 0)))
    zeros1 = jnp.zeros((NP,), jnp.float32)
    zeros2 = jnp.zeros((NP, FB), jnp.float32)

    degs = _deg_call(dst_p, zeros1)
    degt = jnp.transpose(degs)

    g1, dinv = _tc_b_call(embed_p, W1, degt)
    er1 = _gat2(src_p, g1)
    res1 = _sca2(dst_p, er1, zeros2)
    g2 = _tc_d_call(res1, g1, dinv, b1.reshape(1, D_HID), W2)
    er2 = _gat1(src_p, g2)
    res2 = _sca1(dst_p, er2, zeros2)
    out = _tc_f_call(res2, g2, dinv, b2.reshape(1, D_OUT),
                     gamma.reshape(1, D_OUT), beta.reshape(1, D_OUT))
    return out


# speedup vs baseline: 1.4185x; 1.1544x over previous
import functools

import jax
import jax.numpy as jnp
from jax import lax
from jax.experimental import pallas as pl
from jax.experimental.pallas import tpu as pltpu
from jax.experimental.pallas import tpu_sc as plsc

N = 10000
E = 320000
D_IN = 128
D_HID = 256
D_OUT = 128

NC = 2
NS = 16
NW = NC * NS

CH = 128
NP = 10240
ROWS_PER_TILE = NP // NS
EP = 327680
FB = 128
NPS = 10112

_mesh = plsc.VectorSubcoreMesh(core_axis_name="c", subcore_axis_name="s")


def _deg_body(dst_hbm, zeros_hbm, out_hbm, dst_v, ones_v, acc, _sem):
    c = lax.axis_index("c")
    s = lax.axis_index("s")
    r0 = s * ROWS_PER_TILE
    pltpu.sync_copy(zeros_hbm.at[pl.ds(r0, ROWS_PER_TILE)],
                    acc.at[pl.ds(r0, ROWS_PER_TILE)])
    for i in range(CH // 16):
        ones_v[pl.ds(i * 16, 16)] = jnp.ones((16,), jnp.float32)

    nch = EP // NW // CH
    ch0 = (c * NS + s) * nch
    pltpu.sync_copy(dst_hbm.at[pl.ds(ch0, nch)], dst_v)
    plsc.subcore_barrier()

    def body(i, carry):
        pltpu.sync_copy(ones_v, acc.at[dst_v.at[i]], add=True)
        return carry

    lax.fori_loop(0, nch, body, 0)
    plsc.subcore_barrier()
    pltpu.sync_copy(acc.at[pl.ds(r0, ROWS_PER_TILE)],
                    out_hbm.at[c, pl.ds(r0, ROWS_PER_TILE)])


_deg_call = pl.kernel(
    _deg_body,
    out_type=jax.ShapeDtypeStruct((NC, NP), jnp.float32),
    mesh=_mesh,
    scratch_types=[
        pltpu.VMEM((EP // NW // CH, CH), jnp.int32),
        pltpu.VMEM((CH,), jnp.float32),
        pltpu.VMEM_SHARED((NP,), jnp.float32),
        pltpu.SemaphoreType.DMA,
    ],
)


def _gat_body(src_hbm, g_hbm, er_hbm,
              src0, src1, src2, rows0, rows1, rows2, stbl,
              gsem0, gsem1, gsem2, isem0, isem1, isem2,
              wsem0, wsem1, wsem2, *, ncb):
    srcb = [src0, src1, src2]
    rows = [rows0, rows1, rows2]
    gsem = [gsem0, gsem1, gsem2]
    isem = [isem0, isem1, isem2]
    wsem = [wsem0, wsem1, wsem2]
    c = lax.axis_index("c")
    s = lax.axis_index("s")
    rpt = pl.ds(s * ROWS_PER_TILE, ROWS_PER_TILE)
    if ncb == 2:
        nch = EP // NS // CH
        ch0 = s * nch
        kb = c
    else:
        nch = EP // NW // CH
        ch0 = (c * NS + s) * nch
        kb = 0
    rps = pl.ds(s * (NPS // NS), NPS // NS)
    pltpu.sync_copy(g_hbm.at[kb].at[rps], stbl.at[rps])
    plsc.subcore_barrier()

    def _er(ch):
        return er_hbm.at[kb].at[pl.ds((ch0 + ch) * CH, CH)]

    pltpu.sync_copy(src_hbm.at[ch0], srcb[0])
    pltpu.async_copy(src_hbm.at[ch0 + 1], srcb[1], isem[1])
    pltpu.async_copy(src_hbm.at[ch0 + 2], srcb[2], isem[2])
    pltpu.async_copy(stbl.at[srcb[0]], rows[0], gsem[0])

    def body(jj, carry):
        for b in range(3):
            ch = jj * 3 + b
            o = (b + 1) % 3
            pltpu.make_async_copy(stbl.at[srcb[b]], rows[b], gsem[b]).wait()

            @pl.when(ch + 1 < nch)
            def _():
                pltpu.make_async_copy(src_hbm.at[ch0], srcb[o], isem[o]).wait()

                @pl.when(ch >= 2)
                def _():
                    pltpu.make_async_copy(rows[o], _er(0), wsem[o]).wait()

                pltpu.async_copy(stbl.at[srcb[o]], rows[o], gsem[o])

            pltpu.async_copy(rows[b], _er(ch), wsem[b])

            @pl.when(ch + 3 < nch)
            def _():
                pltpu.async_copy(src_hbm.at[ch0 + ch + 3], srcb[b], isem[b])
        return carry

    lax.fori_loop(0, nch // 3, body, 0)
    for ch in range(3 * (nch // 3), nch):
        b = ch % 3
        o = (ch + 1) % 3
        pltpu.make_async_copy(stbl.at[srcb[b]], rows[b], gsem[b]).wait()
        if ch + 1 < nch:
            pltpu.make_async_copy(src_hbm.at[ch0], srcb[o], isem[o]).wait()
            pltpu.make_async_copy(rows[o], _er(0), wsem[o]).wait()
            pltpu.async_copy(stbl.at[srcb[o]], rows[o], gsem[o])
        pltpu.async_copy(rows[b], _er(ch), wsem[b])
    for ch in range(nch - 3, nch):
        pltpu.make_async_copy(rows[ch % 3], _er(0), wsem[ch % 3]).wait()


def _make_gat(ncb):
    return pl.kernel(
        functools.partial(_gat_body, ncb=ncb),
        out_type=jax.ShapeDtypeStruct((ncb, EP, FB), jnp.float32),
        mesh=_mesh,
        scratch_types=[
            pltpu.VMEM((CH,), jnp.int32),
            pltpu.VMEM((CH,), jnp.int32),
            pltpu.VMEM((CH,), jnp.int32),
            pltpu.VMEM((CH, FB), jnp.float32),
            pltpu.VMEM((CH, FB), jnp.float32),
            pltpu.VMEM((CH, FB), jnp.float32),
            pltpu.VMEM_SHARED((NPS, FB), jnp.float32),
            pltpu.SemaphoreType.DMA,
            pltpu.SemaphoreType.DMA,
            pltpu.SemaphoreType.DMA,
            pltpu.SemaphoreType.DMA,
            pltpu.SemaphoreType.DMA,
            pltpu.SemaphoreType.DMA,
            pltpu.SemaphoreType.DMA,
            pltpu.SemaphoreType.DMA,
            pltpu.SemaphoreType.DMA,
        ],
    )


def _sca_body(dst_hbm, er_hbm, zeros_hbm, out_hbm,
              dst0, dst1, rows0, rows1, acc,
              rsem0, rsem1, isem0, isem1, *, ncb):
    dstb = [dst0, dst1]
    rows = [rows0, rows1]
    rsem = [rsem0, rsem1]
    isem = [isem0, isem1]
    c = lax.axis_index("c")
    s = lax.axis_index("s")
    rpt = pl.ds(s * ROWS_PER_TILE, ROWS_PER_TILE)
    if ncb == 2:
        nch = EP // NS // CH
        ch0 = s * nch
        kb = c
    else:
        nch = EP // NW // CH
        ch0 = (c * NS + s) * nch
        kb = 0
    pltpu.sync_copy(zeros_hbm.at[rpt], acc.at[rpt])
    plsc.subcore_barrier()

    def _er(ch):
        return er_hbm.at[kb].at[pl.ds((ch0 + ch) * CH, CH)]

    pltpu.sync_copy(dst_hbm.at[ch0], dstb[0])
    pltpu.async_copy(dst_hbm.at[ch0 + 1], dstb[1], isem[1])
    pltpu.async_copy(_er(0), rows[0], rsem[0])

    def body(jj, carry):
        for b in range(2):
            ch = jj * 2 + b
            o = 1 - b
            pltpu.make_async_copy(_er(ch), rows[b], rsem[b]).wait()

            @pl.when(ch + 1 < nch)
            def _():
                pltpu.async_copy(_er(ch + 1), rows[o], rsem[o])

            @pl.when(ch >= 1)
            def _():
                pltpu.make_async_copy(dst_hbm.at[ch0], dstb[b], isem[b]).wait()

            pltpu.sync_copy(rows[b], acc.at[dstb[b]], add=True)

            @pl.when(ch + 2 < nch)
            def _():
                pltpu.async_copy(dst_hbm.at[ch0 + ch + 2], dstb[b], isem[b])
        return carry

    lax.fori_loop(0, nch // 2, body, 0)
    if nch % 2:
        pltpu.make_async_copy(_er(nch - 1), rows[0], rsem[0]).wait()
        pltpu.make_async_copy(dst_hbm.at[ch0], dstb[0], isem[0]).wait()
        pltpu.sync_copy(rows[0], acc.at[dstb[0]], add=True)
    plsc.subcore_barrier()
    pltpu.sync_copy(acc.at[rpt], out_hbm.at[c].at[rpt])


def _make_sca(ncb):
    return pl.kernel(
        functools.partial(_sca_body, ncb=ncb),
        out_type=jax.ShapeDtypeStruct((NC, NP, FB), jnp.float32),
        mesh=_mesh,
        scratch_types=[
            pltpu.VMEM((CH,), jnp.int32),
            pltpu.VMEM((CH,), jnp.int32),
            pltpu.VMEM((CH, FB), jnp.float32),
            pltpu.VMEM((CH, FB), jnp.float32),
            pltpu.VMEM_SHARED((NP, FB), jnp.float32),
            pltpu.SemaphoreType.DMA,
            pltpu.SemaphoreType.DMA,
            pltpu.SemaphoreType.DMA,
            pltpu.SemaphoreType.DMA,
        ],
    )


_gat2 = _make_gat(2)
_gat1 = _make_gat(1)
_sca2 = _make_sca(2)
_sca1 = _make_sca(1)


def _tc_b(embed_ref, w1_ref, degt_ref, g1_ref, dinv_ref):
    degt = degt_ref[...]
    deg = degt[:, 0:1] + degt[:, 1:2] + 1.0
    dinv = lax.rsqrt(deg)
    h = jnp.dot(embed_ref[...], w1_ref[...],
                preferred_element_type=jnp.float32)
    g = h * dinv
    for k in range(D_HID // FB):
        g1_ref[k] = g[:, k * FB:(k + 1) * FB]
    dinv_ref[...] = dinv


def _tc_d(res1_ref, g1_ref, dinv_ref, b1_ref, w2_ref, g2_ref):
    dinv = dinv_ref[...]
    b1 = b1_ref[...]
    w2 = w2_ref[...]
    acc = jnp.zeros((res1_ref.shape[1], D_OUT), jnp.float32)
    for k in range(D_HID // FB):
        hk = (res1_ref[k] + g1_ref[k]) * dinv + b1[:, k * FB:(k + 1) * FB]
        acc = acc + jnp.dot(hk, w2[k * FB:(k + 1) * FB],
                            preferred_element_type=jnp.float32)
    g2 = acc * dinv
    for k in range(D_OUT // FB):
        g2_ref[k] = g2[:, k * FB:(k + 1) * FB]


def _tc_f(res2_ref, g2_ref, dinv_ref, b2_ref, gamma_ref, beta_ref, out_ref):
    o = res2_ref[0] + res2_ref[1] + g2_ref[0]
    o = o * dinv_ref[...] + b2_ref[...]
    rowid = lax.broadcasted_iota(jnp.int32, (NP, 1), 0)
    mask = (rowid < N).astype(jnp.float32)
    mu = jnp.sum(o * mask, axis=0, keepdims=True) * (1.0 / N)
    d = (o - mu) * mask
    var = jnp.sum(d * d, axis=0, keepdims=True) * (1.0 / N)
    y = (o - mu) * lax.rsqrt(var + 1e-5) * gamma_ref[...] + beta_ref[...]
    out_ref[...] = y[:N]


_RB = 2048

_tc_b_call = pl.pallas_call(
    _tc_b,
    grid=(NP // _RB,),
    in_specs=[pl.BlockSpec((_RB, D_IN), lambda i: (i, 0)),
              pl.BlockSpec((D_IN, D_HID), lambda i: (0, 0)),
              pl.BlockSpec((_RB, NC), lambda i: (i, 0))],
    out_specs=(pl.BlockSpec((D_HID // FB, _RB, FB), lambda i: (0, i, 0)),
               pl.BlockSpec((_RB, 1), lambda i: (i, 0))),
    out_shape=(jax.ShapeDtypeStruct((D_HID // FB, NP, FB), jnp.float32),
               jax.ShapeDtypeStruct((NP, 1), jnp.float32)),
)

_tc_d_call = pl.pallas_call(
    _tc_d,
    grid=(NP // _RB,),
    in_specs=[pl.BlockSpec((D_HID // FB, _RB, FB), lambda i: (0, i, 0)),
              pl.BlockSpec((D_HID // FB, _RB, FB), lambda i: (0, i, 0)),
              pl.BlockSpec((_RB, 1), lambda i: (i, 0)),
              pl.BlockSpec((1, D_HID), lambda i: (0, 0)),
              pl.BlockSpec((D_HID, D_OUT), lambda i: (0, 0))],
    out_specs=pl.BlockSpec((D_OUT // FB, _RB, FB), lambda i: (0, i, 0)),
    out_shape=jax.ShapeDtypeStruct((D_OUT // FB, NP, FB), jnp.float32),
)

_tc_f_call = pl.pallas_call(
    _tc_f,
    out_shape=jax.ShapeDtypeStruct((N, D_OUT), jnp.float32),
)


@jax.jit
def kernel(embed, edge_index, W1, b1, W2, b2, gamma, beta):
    src = edge_index[0]
    dst = edge_index[1]
    pad_idx = jnp.full((EP - E,), N, dtype=jnp.int32)
    src_p = jnp.concatenate([src, pad_idx]).reshape(EP // CH, CH)
    dst_p = jnp.concatenate([dst, pad_idx]).reshape(EP // CH, CH)
    embed_p = jnp.pad(embed, ((0, NP - N), (0, 0)))
    zeros1 = jnp.zeros((NP,), jnp.float32)
    zeros2 = jnp.zeros((NP, FB), jnp.float32)

    degs = _deg_call(dst_p, zeros1)
    degt = jnp.transpose(degs)

    g1, dinv = _tc_b_call(embed_p, W1, degt)
    er1 = _gat2(src_p, g1)
    res1 = _sca2(dst_p, er1, zeros2)
    g2 = _tc_d_call(res1, g1, dinv, b1.reshape(1, D_HID), W2)
    er2 = _gat1(src_p, g2)
    res2 = _sca1(dst_p, er2, zeros2)
    out = _tc_f_call(res2, g2, dinv, b2.reshape(1, D_OUT),
                     gamma.reshape(1, D_OUT), beta.reshape(1, D_OUT))
    return out
